# baseline (speedup 1.0000x reference)
import functools

import jax
import jax.numpy as jnp
import numpy as np
from jax import lax
from jax.experimental import pallas as pl
from jax.experimental.pallas import tpu as pltpu
from jax.experimental.pallas import tpu_sc as plsc

NC = 2
NS = 16
L = 16
NW = NC * NS

BATCH = 80
BR = 2000


def _interleave_perm(f):
    idx = np.zeros((f,), np.int64)
    for k in range(f // 32):
        for s in range(32):
            i = s // 2
            idx[32 * k + s] = 32 * k + (i if s % 2 == 0 else L + i)
    p = np.zeros((f, f), np.float32)
    p[idx, np.arange(f)] = 1.0
    return jnp.asarray(p)


def _sc_counts(rowv, colv, typv, *, n_nodes, n_rel, epw, cnt_pad):
    nb = epw // BATCH
    gpb = BATCH // L
    stripe = cnt_pad // NS
    assert nb % 2 == 1 and nb >= 3
    mesh = plsc.VectorSubcoreMesh(
        core_axis_name="c", subcore_axis_name="s",
        num_cores=NC, num_subcores=NS)

    @functools.partial(
        pl.kernel,
        out_type=jax.ShapeDtypeStruct((NC * cnt_pad,), jnp.float32),
        mesh=mesh,
        scratch_types=[
            pltpu.VMEM((epw,), jnp.int32),
            pltpu.VMEM((epw,), jnp.int32),
            pltpu.VMEM((epw,), jnp.int32),
            [pltpu.VMEM((BATCH,), jnp.int32) for _ in range(2)],
            [pltpu.VMEM((BATCH,), jnp.int32) for _ in range(2)],
            pltpu.VMEM((BATCH,), jnp.float32),
            pltpu.VMEM((stripe,), jnp.float32),
            pltpu.VMEM_SHARED((cnt_pad,), jnp.float32),
            [pltpu.SemaphoreType.DMA for _ in range(2)],
        ],
        compiler_params=pltpu.CompilerParams(use_tc_tiling_on_sc=False,
                                             needs_layout_passes=False),
    )
    def k(row_hbm, col_hbm, typ_hbm, out_hbm,
          rv, cv, tv, civs, divs, ones, zv, acc, sems):
        cid = lax.axis_index("c")
        sid = lax.axis_index("s")
        wid = sid * NC + cid
        base = wid * epw

        def zstep(i, carry):
            zv[pl.ds(i * L, L)] = jnp.zeros((L,), jnp.float32)
            return carry

        lax.fori_loop(0, stripe // L, zstep, 0)
        pltpu.sync_copy(zv, acc.at[pl.ds(sid * stripe, stripe)])
        for gq in range(gpb):
            ones[pl.ds(gq * L, L)] = jnp.ones((L,), jnp.float32)
        pltpu.sync_copy(row_hbm.at[pl.ds(base, epw)], rv)
        pltpu.sync_copy(col_hbm.at[pl.ds(base, epw)], cv)
        pltpu.sync_copy(typ_hbm.at[pl.ds(base, epw)], tv)
        plsc.subcore_barrier()

        def start(b, p):
            off = b * BATCH
            for gq in range(gpb):
                t = tv[pl.ds(off + gq * L, L)]
                cc = cv[pl.ds(off + gq * L, L)]
                rr = rv[pl.ds(off + gq * L, L)]
                civs[p][pl.ds(gq * L, L)] = t * n_nodes + cc
                divs[p][pl.ds(gq * L, L)] = t * n_nodes + rr + n_rel * n_nodes
            pltpu.async_copy(ones, acc.at[civs[p]], sems[p], add=True)
            pltpu.async_copy(ones, acc.at[divs[p]], sems[p], add=True)

        def finish(p):
            pltpu.make_async_copy(ones, acc.at[civs[p]], sems[p]).wait()
            pltpu.make_async_copy(ones, acc.at[divs[p]], sems[p]).wait()

        start(0, 0)

        def bstep(i, carry):
            b0 = 2 * i
            start(b0 + 1, 1)
            finish(0)
            start(b0 + 2, 0)
            finish(1)
            return carry

        lax.fori_loop(0, (nb - 1) // 2, bstep, 0)
        finish(0)
        plsc.subcore_barrier()
        pltpu.sync_copy(acc.at[pl.ds(sid * stripe, stripe)], zv)
        pltpu.sync_copy(zv,
                        out_hbm.at[pl.ds(cid * cnt_pad + sid * stripe, stripe)])

    return k(rowv, colv, typv)


def _sc_edge_pass(rowv, colv, typv, v3, gtab, *, n_nodes, epw, n_pad, f):
    chk = 2000
    bpc = chk // BATCH
    nb = epw // BATCH
    gpb = BATCH // L
    astripe = n_pad // NS
    rn = v3.shape[0]
    assert nb % 2 == 1 and nb >= 3
    g2 = jnp.broadcast_to(gtab.reshape(rn, 1), (rn, L))
    mesh = plsc.VectorSubcoreMesh(
        core_axis_name="c", subcore_axis_name="s",
        num_cores=NC, num_subcores=NS)

    @functools.partial(
        pl.kernel,
        out_type=jax.ShapeDtypeStruct((NC, n_pad, f), jnp.float32),
        mesh=mesh,
        scratch_types=[
            pltpu.VMEM((chk,), jnp.int32),
            pltpu.VMEM((chk,), jnp.int32),
            pltpu.VMEM((chk,), jnp.int32),
            [pltpu.VMEM((BATCH,), jnp.int32) for _ in range(2)],
            [pltpu.VMEM((BATCH,), jnp.int32) for _ in range(2)],
            [pltpu.VMEM((BATCH,), jnp.int32) for _ in range(2)],
            [pltpu.VMEM((BATCH, L), jnp.float32) for _ in range(2)],
            [pltpu.VMEM((BATCH, f), jnp.bfloat16) for _ in range(2)],
            pltpu.VMEM((BATCH, f), jnp.float32),
            pltpu.VMEM_SHARED((n_pad, f), jnp.float32),
            [pltpu.SemaphoreType.DMA for _ in range(2)],
            [pltpu.SemaphoreType.DMA for _ in range(2)],
        ],
        compiler_params=pltpu.CompilerParams(use_tc_tiling_on_sc=False,
                                             needs_layout_passes=False),
    )
    def k(row_hbm, col_hbm, typ_hbm, v3_hbm, g_hbm, out_hbm,
          rv, cv, tv, srcvs, dstvs, widxs, wbs, rowsbf, rowsf, acc,
          semvs, semws):
        cid = lax.axis_index("c")
        sid = lax.axis_index("s")
        wid = sid * NC + cid
        base = wid * epw
        rows0 = rowsf

        def zrow(j, carry):
            for kq in range(f // L):
                rows0[j, pl.ds(kq * L, L)] = jnp.zeros((L,), jnp.float32)
            return carry

        lax.fori_loop(0, BATCH, zrow, 0)
        for ch in range(astripe // BATCH):
            pltpu.sync_copy(
                rows0, acc.at[pl.ds(sid * astripe + ch * BATCH, BATCH)])
        plsc.subcore_barrier()

        def start(b, p):
            @pl.when(lax.rem(b, bpc) == 0)
            def _():
                coff = base + (b // bpc) * chk
                pltpu.sync_copy(row_hbm.at[pl.ds(coff, chk)], rv)
                pltpu.sync_copy(col_hbm.at[pl.ds(coff, chk)], cv)
                pltpu.sync_copy(typ_hbm.at[pl.ds(coff, chk)], tv)
            off = lax.rem(b, bpc) * BATCH
            for gq in range(gpb):
                t = tv[pl.ds(off + gq * L, L)]
                cc = cv[pl.ds(off + gq * L, L)]
                rr = rv[pl.ds(off + gq * L, L)]
                srcvs[p][pl.ds(gq * L, L)] = t * n_nodes + cc
                dstvs[p][pl.ds(gq * L, L)] = rr
                widxs[p][pl.ds(gq * L, L)] = t * n_nodes + rr
            pltpu.async_copy(v3_hbm.at[srcvs[p]], rowsbf[p], semvs[p])
            pltpu.async_copy(g_hbm.at[widxs[p]], wbs[p], semws[p])

        def finish(p):
            pltpu.make_async_copy(v3_hbm.at[srcvs[p]], rowsbf[p],
                                  semvs[p]).wait()
            pltpu.make_async_copy(g_hbm.at[widxs[p]], wbs[p],
                                  semws[p]).wait()
            rbf = rowsbf[p]
            wb = wbs[p]

            def escale(jj, c2):
                for u in range(4):
                    j = jj * 4 + u
                    wrep = wb[j, pl.ds(0, L)]
                    for kq in range(f // (2 * L)):
                        v32 = rbf[j, pl.ds(kq * 2 * L, 2 * L)]
                        va, vb = plsc.unpack(
                            v32, format=plsc.PackFormat.INTERLEAVED)
                        rowsf[j, pl.ds(kq * 2 * L, L)] = va * wrep
                        rowsf[j, pl.ds(kq * 2 * L + L, L)] = vb * wrep
                return c2

            lax.fori_loop(0, BATCH // 4, escale, 0)
            pltpu.sync_copy(rowsf, acc.at[dstvs[p]], add=True)

        start(0, 0)

        def bstep(i, carry):
            b0 = 2 * i
            start(b0 + 1, 1)
            finish(0)
            start(b0 + 2, 0)
            finish(1)
            return carry

        lax.fori_loop(0, (nb - 1) // 2, bstep, 0)
        finish(0)
        plsc.subcore_barrier()
        for ch in range(astripe // BATCH):
            off2 = sid * astripe + ch * BATCH
            pltpu.sync_copy(acc.at[pl.ds(off2, BATCH)], rows0)
            pltpu.sync_copy(rows0, out_hbm.at[cid, pl.ds(off2, BATCH)])

    return k(rowv, colv, typv, v3, g2)


def _tc_reduce(x, cd, *, n_rel, f):
    n = x.shape[0]
    nblk = n // BR

    def body(xb, cb, s_ref, aux_ref):
        pid = pl.program_id(0)

        @pl.when(pid == 0)
        def _():
            s_ref[...] = jnp.zeros_like(s_ref)
            aux_ref[...] = jnp.zeros_like(aux_ref)

        xv = xb[...]
        cv = cb[...]
        ones_row = jnp.ones((1, f), jnp.float32)
        for r in range(n_rel):
            cr = cv[:, r:r + 1] + cv[:, n_rel + r:n_rel + r + 1]
            cr128 = lax.dot_general(cr, ones_row, (((1,), (0,)), ((), ())),
                                    preferred_element_type=jnp.float32)
            xc = xv * cr128
            s_ref[r] += lax.dot_general(
                xc, xv, (((0,), (0,)), ((), ())),
                preferred_element_type=jnp.float32)
            s_r = jnp.sum(xc, axis=0, keepdims=True)
            cnt_row = jnp.sum(cr128, axis=0, keepdims=True)
            upd = jnp.concatenate(
                [s_r, cnt_row, jnp.zeros((6, f), jnp.float32)], axis=0)
            aux_ref[r] += upd

    return pl.pallas_call(
        body,
        grid=(nblk,),
        in_specs=[
            pl.BlockSpec((BR, f), lambda i: (i, 0)),
            pl.BlockSpec((BR, NC * n_rel), lambda i: (i, 0)),
        ],
        out_specs=[
            pl.BlockSpec((n_rel, f, f), lambda i: (0, 0, 0)),
            pl.BlockSpec((n_rel, 8, f), lambda i: (0, 0, 0)),
        ],
        out_shape=[
            jax.ShapeDtypeStruct((n_rel, f, f), jnp.float32),
            jax.ShapeDtypeStruct((n_rel, 8, f), jnp.float32),
        ],
    )(x, cd)


def _tc_prepare(x, dd, s3, aux, wq, wk, wv, wp, pmat, *, n_rel, f):
    n = x.shape[0]
    nblk = n // BR

    def body(xb, db, s_ref, aux_ref, wq_ref, wk_ref, wv_ref, wp_ref, pm_ref,
             v3_ref, g_ref, dense_ref):
        xv = xb[...]
        dv = db[...]
        wp_ = wp_ref[...]
        ones_row = jnp.ones((1, f), jnp.float32)
        dense = jnp.zeros((BR, f), jnp.float32)
        for r in range(n_rel):
            wqr = wq_ref[r]
            wkr = wk_ref[r]
            wvr = wv_ref[r]
            s_mat = s_ref[r]
            sv = lax.dot_general(s_mat, wvr, (((1,), (0,)), ((), ())),
                                 preferred_element_type=jnp.float32)
            kvs = lax.dot_general(wkr, sv, (((0,), (0,)), ((), ())),
                                  preferred_element_type=jnp.float32)
            a2 = wqr @ kvs @ wp_
            s_row = aux_ref[r, 0:1, :]
            ks_row = lax.dot_general(s_row, wkr, (((1,), (0,)), ((), ())),
                                     preferred_element_type=jnp.float32)
            b_row = lax.dot_general(ks_row, wqr, (((1,), (1,)), ((), ())),
                                    preferred_element_type=jnp.float32)
            b_mat = lax.dot_general(b_row, ones_row, (((0,), (0,)), ((), ())),
                                    preferred_element_type=jnp.float32)
            cnt_row = aux_ref[r, 1:2, :]
            den = (xv @ b_mat) + cnt_row
            den = jnp.where(den == 0.0, 1.0, den)
            g_val = cnt_row / den
            g_ref[:, r:r + 1] = g_val[:, 0:1]
            v3_ref[r] = (xv @ ((wvr @ wp_) @ pm_ref[...])).astype(jnp.bfloat16)
            dr = dv[:, r:r + 1] + dv[:, n_rel + r:n_rel + r + 1]
            dr128 = lax.dot_general(dr, ones_row, (((1,), (0,)), ((), ())),
                                    preferred_element_type=jnp.float32)
            dense = dense + (xv @ a2) * (dr128 / den)
        dense_ref[...] = dense

    return pl.pallas_call(
        body,
        grid=(nblk,),
        in_specs=[
            pl.BlockSpec((BR, f), lambda i: (i, 0)),
            pl.BlockSpec((BR, NC * n_rel), lambda i: (i, 0)),
            pl.BlockSpec((n_rel, f, f), lambda i: (0, 0, 0)),
            pl.BlockSpec((n_rel, 8, f), lambda i: (0, 0, 0)),
            pl.BlockSpec((n_rel, f, f), lambda i: (0, 0, 0)),
            pl.BlockSpec((n_rel, f, f), lambda i: (0, 0, 0)),
            pl.BlockSpec((n_rel, f, f), lambda i: (0, 0, 0)),
            pl.BlockSpec((f, f), lambda i: (0, 0)),
            pl.BlockSpec((f, f), lambda i: (0, 0)),
        ],
        out_specs=[
            pl.BlockSpec((n_rel, BR, f), lambda i: (0, i, 0)),
            pl.BlockSpec((BR, n_rel), lambda i: (i, 0)),
            pl.BlockSpec((BR, f), lambda i: (i, 0)),
        ],
        out_shape=[
            jax.ShapeDtypeStruct((n_rel, n, f), jnp.bfloat16),
            jax.ShapeDtypeStruct((n, n_rel), jnp.float32),
            jax.ShapeDtypeStruct((n, f), jnp.float32),
        ],
    )(x, dd, s3, aux, wq, wk, wv, wp, pmat)


def _tc_combine(parts, dense, *, f):
    n = dense.shape[0]
    nblk = n // BR

    def body(p_ref, d_ref, o_ref):
        o_ref[...] = p_ref[0] + p_ref[1] + d_ref[...]

    return pl.pallas_call(
        body,
        grid=(nblk,),
        in_specs=[
            pl.BlockSpec((NC, BR, f), lambda i: (0, i, 0)),
            pl.BlockSpec((BR, f), lambda i: (i, 0)),
        ],
        out_specs=pl.BlockSpec((BR, f), lambda i: (i, 0)),
        out_shape=jax.ShapeDtypeStruct((n, f), jnp.float32),
    )(parts, dense)


def kernel(x, edge_index, edge_type, Wq, Wk, Wv, Wp):
    n, f = x.shape
    r_ = Wq.shape[0]
    e = edge_type.shape[0]
    assert e % NW == 0
    epw = e // NW
    assert epw % BATCH == 0 and f % L == 0 and n % BR == 0

    rn = r_ * n
    cnt_pad = ((2 * rn + NS * L - 1) // (NS * L)) * (NS * L)
    n_pad = ((n + NS * BATCH - 1) // (NS * BATCH)) * (NS * BATCH)
    assert r_ <= 4

    rowv = edge_index[0]
    colv = edge_index[1]
    typv = edge_type.astype(jnp.int32)

    cnts = _sc_counts(rowv, colv, typv,
                      n_nodes=n, n_rel=r_, epw=epw,
                      cnt_pad=cnt_pad).reshape(NC, cnt_pad)
    cd = cnts[:, :rn].reshape(NC * r_, n).T
    dd = cnts[:, rn:2 * rn].reshape(NC * r_, n).T

    s3, aux = _tc_reduce(x, cd, n_rel=r_, f=f)
    v3, g, dense = _tc_prepare(x, dd, s3, aux, Wq, Wk, Wv, Wp,
                               _interleave_perm(f), n_rel=r_, f=f)

    gtab = g.T.reshape(rn)
    parts = _sc_edge_pass(rowv, colv, typv,
                          v3.reshape(rn, f), gtab,
                          n_nodes=n, epw=epw, n_pad=n_pad, f=f)

    return _tc_combine(parts, dense, f=f)


# speedup vs baseline: 1.0013x; 1.0013x over previous
import functools

import jax
import jax.numpy as jnp
import numpy as np
from jax import lax
from jax.experimental import pallas as pl
from jax.experimental.pallas import tpu as pltpu
from jax.experimental.pallas import tpu_sc as plsc

NC = 2
NS = 16
L = 16
NW = NC * NS

BATCH = 80
BR = 2000


def _interleave_perm(f):
    idx = np.zeros((f,), np.int64)
    for k in range(f // 32):
        for s in range(32):
            i = s // 2
            idx[32 * k + s] = 32 * k + (i if s % 2 == 0 else L + i)
    p = np.zeros((f, f), np.float32)
    p[idx, np.arange(f)] = 1.0
    return jnp.asarray(p)


def _sc_counts(rowv, colv, typv, *, n_nodes, n_rel, epw, cnt_pad):
    nb = epw // BATCH
    gpb = BATCH // L
    stripe = cnt_pad // NS
    assert nb % 2 == 1 and nb >= 3
    mesh = plsc.VectorSubcoreMesh(
        core_axis_name="c", subcore_axis_name="s",
        num_cores=NC, num_subcores=NS)

    @functools.partial(
        pl.kernel,
        out_type=jax.ShapeDtypeStruct((NC * cnt_pad,), jnp.float32),
        mesh=mesh,
        scratch_types=[
            pltpu.VMEM((epw,), jnp.int32),
            pltpu.VMEM((epw,), jnp.int32),
            pltpu.VMEM((epw,), jnp.int32),
            [pltpu.VMEM((BATCH,), jnp.int32) for _ in range(2)],
            [pltpu.VMEM((BATCH,), jnp.int32) for _ in range(2)],
            pltpu.VMEM((BATCH,), jnp.float32),
            pltpu.VMEM((stripe,), jnp.float32),
            pltpu.VMEM_SHARED((cnt_pad,), jnp.float32),
            [pltpu.SemaphoreType.DMA for _ in range(2)],
        ],
        compiler_params=pltpu.CompilerParams(use_tc_tiling_on_sc=False,
                                             needs_layout_passes=False),
    )
    def k(row_hbm, col_hbm, typ_hbm, out_hbm,
          rv, cv, tv, civs, divs, ones, zv, acc, sems):
        cid = lax.axis_index("c")
        sid = lax.axis_index("s")
        wid = sid * NC + cid
        base = wid * epw

        def zstep(i, carry):
            zv[pl.ds(i * L, L)] = jnp.zeros((L,), jnp.float32)
            return carry

        lax.fori_loop(0, stripe // L, zstep, 0)
        pltpu.sync_copy(zv, acc.at[pl.ds(sid * stripe, stripe)])
        for gq in range(gpb):
            ones[pl.ds(gq * L, L)] = jnp.ones((L,), jnp.float32)
        pltpu.sync_copy(row_hbm.at[pl.ds(base, epw)], rv)
        pltpu.sync_copy(col_hbm.at[pl.ds(base, epw)], cv)
        pltpu.sync_copy(typ_hbm.at[pl.ds(base, epw)], tv)
        plsc.subcore_barrier()

        def start(b, p):
            off = b * BATCH
            for gq in range(gpb):
                t = tv[pl.ds(off + gq * L, L)]
                cc = cv[pl.ds(off + gq * L, L)]
                rr = rv[pl.ds(off + gq * L, L)]
                civs[p][pl.ds(gq * L, L)] = t * n_nodes + cc
                divs[p][pl.ds(gq * L, L)] = t * n_nodes + rr + n_rel * n_nodes
            pltpu.async_copy(ones, acc.at[civs[p]], sems[p], add=True)
            pltpu.async_copy(ones, acc.at[divs[p]], sems[p], add=True)

        def finish(p):
            pltpu.make_async_copy(ones, acc.at[civs[p]], sems[p]).wait()
            pltpu.make_async_copy(ones, acc.at[divs[p]], sems[p]).wait()

        start(0, 0)

        def bstep(i, carry):
            b0 = 2 * i
            start(b0 + 1, 1)
            finish(0)
            start(b0 + 2, 0)
            finish(1)
            return carry

        lax.fori_loop(0, (nb - 1) // 2, bstep, 0)
        finish(0)
        plsc.subcore_barrier()
        pltpu.sync_copy(acc.at[pl.ds(sid * stripe, stripe)], zv)
        pltpu.sync_copy(zv,
                        out_hbm.at[pl.ds(cid * cnt_pad + sid * stripe, stripe)])

    return k(rowv, colv, typv)


def _sc_edge_pass(rowv, colv, typv, v3, gtab, *, n_nodes, epw, n_pad, f):
    chk = 2000
    bpc = chk // BATCH
    nb = epw // BATCH
    gpb = BATCH // L
    astripe = n_pad // NS
    rn = v3.shape[0]
    assert nb % 2 == 1 and nb >= 3
    g2 = jnp.broadcast_to(gtab.reshape(rn, 1), (rn, L))
    mesh = plsc.VectorSubcoreMesh(
        core_axis_name="c", subcore_axis_name="s",
        num_cores=NC, num_subcores=NS)

    @functools.partial(
        pl.kernel,
        out_type=jax.ShapeDtypeStruct((NC, n_pad, f), jnp.float32),
        mesh=mesh,
        scratch_types=[
            pltpu.VMEM((chk,), jnp.int32),
            pltpu.VMEM((chk,), jnp.int32),
            pltpu.VMEM((chk,), jnp.int32),
            [pltpu.VMEM((BATCH,), jnp.int32) for _ in range(2)],
            [pltpu.VMEM((BATCH,), jnp.int32) for _ in range(2)],
            [pltpu.VMEM((BATCH,), jnp.int32) for _ in range(2)],
            [pltpu.VMEM((BATCH, L), jnp.float32) for _ in range(2)],
            [pltpu.VMEM((BATCH, f), jnp.bfloat16) for _ in range(2)],
            pltpu.VMEM((BATCH, f), jnp.float32),
            pltpu.VMEM_SHARED((n_pad, f), jnp.float32),
            [pltpu.SemaphoreType.DMA for _ in range(2)],
            [pltpu.SemaphoreType.DMA for _ in range(2)],
        ],
        compiler_params=pltpu.CompilerParams(use_tc_tiling_on_sc=False,
                                             needs_layout_passes=False),
    )
    def k(row_hbm, col_hbm, typ_hbm, v3_hbm, g_hbm, out_hbm,
          rv, cv, tv, srcvs, dstvs, widxs, wbs, rowsbf, rowsf, acc,
          semvs, semws):
        cid = lax.axis_index("c")
        sid = lax.axis_index("s")
        wid = sid * NC + cid
        base = wid * epw
        rows0 = rowsf

        def zrow(j, carry):
            for kq in range(f // L):
                rows0[j, pl.ds(kq * L, L)] = jnp.zeros((L,), jnp.float32)
            return carry

        lax.fori_loop(0, BATCH, zrow, 0)
        for ch in range(astripe // BATCH):
            pltpu.sync_copy(
                rows0, acc.at[pl.ds(sid * astripe + ch * BATCH, BATCH)])
        plsc.subcore_barrier()

        def start(b, p):
            @pl.when(lax.rem(b, bpc) == 0)
            def _():
                coff = base + (b // bpc) * chk
                pltpu.sync_copy(row_hbm.at[pl.ds(coff, chk)], rv)
                pltpu.sync_copy(col_hbm.at[pl.ds(coff, chk)], cv)
                pltpu.sync_copy(typ_hbm.at[pl.ds(coff, chk)], tv)
            off = lax.rem(b, bpc) * BATCH
            for gq in range(gpb):
                t = tv[pl.ds(off + gq * L, L)]
                cc = cv[pl.ds(off + gq * L, L)]
                rr = rv[pl.ds(off + gq * L, L)]
                srcvs[p][pl.ds(gq * L, L)] = t * n_nodes + cc
                dstvs[p][pl.ds(gq * L, L)] = rr
                widxs[p][pl.ds(gq * L, L)] = t * n_nodes + rr
            pltpu.async_copy(v3_hbm.at[srcvs[p]], rowsbf[p], semvs[p])
            pltpu.async_copy(g_hbm.at[widxs[p]], wbs[p], semws[p])

        def finish(p):
            pltpu.make_async_copy(v3_hbm.at[srcvs[p]], rowsbf[p],
                                  semvs[p]).wait()
            pltpu.make_async_copy(g_hbm.at[widxs[p]], wbs[p],
                                  semws[p]).wait()
            rbf = rowsbf[p]
            wb = wbs[p]

            himask = jnp.full((L,), -65536, jnp.int32)

            def escale(jj, c2):
                for u in range(4):
                    j = jj * 4 + u
                    wrep = wb[j, pl.ds(0, L)]
                    for kq in range(f // (2 * L)):
                        v32 = rbf[j, pl.ds(kq * 2 * L, 2 * L)]
                        xi = plsc.bitcast(v32, jnp.int32)
                        va = plsc.bitcast(xi << 16, jnp.float32)
                        vb = plsc.bitcast(xi & himask, jnp.float32)
                        rowsf[j, pl.ds(kq * 2 * L, L)] = va * wrep
                        rowsf[j, pl.ds(kq * 2 * L + L, L)] = vb * wrep
                return c2

            lax.fori_loop(0, BATCH // 4, escale, 0)
            pltpu.sync_copy(rowsf, acc.at[dstvs[p]], add=True)

        start(0, 0)

        def bstep(i, carry):
            b0 = 2 * i
            start(b0 + 1, 1)
            finish(0)
            start(b0 + 2, 0)
            finish(1)
            return carry

        lax.fori_loop(0, (nb - 1) // 2, bstep, 0)
        finish(0)
        plsc.subcore_barrier()
        for ch in range(astripe // BATCH):
            off2 = sid * astripe + ch * BATCH
            pltpu.sync_copy(acc.at[pl.ds(off2, BATCH)], rows0)
            pltpu.sync_copy(rows0, out_hbm.at[cid, pl.ds(off2, BATCH)])

    return k(rowv, colv, typv, v3, g2)


def _tc_reduce(x, cd, *, n_rel, f):
    n = x.shape[0]
    nblk = n // BR

    def body(xb, cb, s_ref, aux_ref):
        pid = pl.program_id(0)

        @pl.when(pid == 0)
        def _():
            s_ref[...] = jnp.zeros_like(s_ref)
            aux_ref[...] = jnp.zeros_like(aux_ref)

        xv = xb[...]
        cv = cb[...]
        ones_row = jnp.ones((1, f), jnp.float32)
        for r in range(n_rel):
            cr = cv[:, r:r + 1] + cv[:, n_rel + r:n_rel + r + 1]
            cr128 = lax.dot_general(cr, ones_row, (((1,), (0,)), ((), ())),
                                    preferred_element_type=jnp.float32)
            xc = xv * cr128
            s_ref[r] += lax.dot_general(
                xc, xv, (((0,), (0,)), ((), ())),
                preferred_element_type=jnp.float32)
            s_r = jnp.sum(xc, axis=0, keepdims=True)
            cnt_row = jnp.sum(cr128, axis=0, keepdims=True)
            upd = jnp.concatenate(
                [s_r, cnt_row, jnp.zeros((6, f), jnp.float32)], axis=0)
            aux_ref[r] += upd

    return pl.pallas_call(
        body,
        grid=(nblk,),
        in_specs=[
            pl.BlockSpec((BR, f), lambda i: (i, 0)),
            pl.BlockSpec((BR, NC * n_rel), lambda i: (i, 0)),
        ],
        out_specs=[
            pl.BlockSpec((n_rel, f, f), lambda i: (0, 0, 0)),
            pl.BlockSpec((n_rel, 8, f), lambda i: (0, 0, 0)),
        ],
        out_shape=[
            jax.ShapeDtypeStruct((n_rel, f, f), jnp.float32),
            jax.ShapeDtypeStruct((n_rel, 8, f), jnp.float32),
        ],
    )(x, cd)


def _tc_prepare(x, dd, s3, aux, wq, wk, wv, wp, pmat, *, n_rel, f):
    n = x.shape[0]
    nblk = n // BR

    def body(xb, db, s_ref, aux_ref, wq_ref, wk_ref, wv_ref, wp_ref, pm_ref,
             v3_ref, g_ref, dense_ref):
        xv = xb[...]
        dv = db[...]
        wp_ = wp_ref[...]
        ones_row = jnp.ones((1, f), jnp.float32)
        dense = jnp.zeros((BR, f), jnp.float32)
        for r in range(n_rel):
            wqr = wq_ref[r]
            wkr = wk_ref[r]
            wvr = wv_ref[r]
            s_mat = s_ref[r]
            sv = lax.dot_general(s_mat, wvr, (((1,), (0,)), ((), ())),
                                 preferred_element_type=jnp.float32)
            kvs = lax.dot_general(wkr, sv, (((0,), (0,)), ((), ())),
                                  preferred_element_type=jnp.float32)
            a2 = wqr @ kvs @ wp_
            s_row = aux_ref[r, 0:1, :]
            ks_row = lax.dot_general(s_row, wkr, (((1,), (0,)), ((), ())),
                                     preferred_element_type=jnp.float32)
            b_row = lax.dot_general(ks_row, wqr, (((1,), (1,)), ((), ())),
                                    preferred_element_type=jnp.float32)
            b_mat = lax.dot_general(b_row, ones_row, (((0,), (0,)), ((), ())),
                                    preferred_element_type=jnp.float32)
            cnt_row = aux_ref[r, 1:2, :]
            den = (xv @ b_mat) + cnt_row
            den = jnp.where(den == 0.0, 1.0, den)
            g_val = cnt_row / den
            g_ref[:, r:r + 1] = g_val[:, 0:1]
            v3_ref[r] = (xv @ ((wvr @ wp_) @ pm_ref[...])).astype(jnp.bfloat16)
            dr = dv[:, r:r + 1] + dv[:, n_rel + r:n_rel + r + 1]
            dr128 = lax.dot_general(dr, ones_row, (((1,), (0,)), ((), ())),
                                    preferred_element_type=jnp.float32)
            dense = dense + (xv @ a2) * (dr128 / den)
        dense_ref[...] = dense

    return pl.pallas_call(
        body,
        grid=(nblk,),
        in_specs=[
            pl.BlockSpec((BR, f), lambda i: (i, 0)),
            pl.BlockSpec((BR, NC * n_rel), lambda i: (i, 0)),
            pl.BlockSpec((n_rel, f, f), lambda i: (0, 0, 0)),
            pl.BlockSpec((n_rel, 8, f), lambda i: (0, 0, 0)),
            pl.BlockSpec((n_rel, f, f), lambda i: (0, 0, 0)),
            pl.BlockSpec((n_rel, f, f), lambda i: (0, 0, 0)),
            pl.BlockSpec((n_rel, f, f), lambda i: (0, 0, 0)),
            pl.BlockSpec((f, f), lambda i: (0, 0)),
            pl.BlockSpec((f, f), lambda i: (0, 0)),
        ],
        out_specs=[
            pl.BlockSpec((n_rel, BR, f), lambda i: (0, i, 0)),
            pl.BlockSpec((BR, n_rel), lambda i: (i, 0)),
            pl.BlockSpec((BR, f), lambda i: (i, 0)),
        ],
        out_shape=[
            jax.ShapeDtypeStruct((n_rel, n, f), jnp.bfloat16),
            jax.ShapeDtypeStruct((n, n_rel), jnp.float32),
            jax.ShapeDtypeStruct((n, f), jnp.float32),
        ],
    )(x, dd, s3, aux, wq, wk, wv, wp, pmat)


def _tc_combine(parts, dense, *, f):
    n = dense.shape[0]
    nblk = n // BR

    def body(p_ref, d_ref, o_ref):
        o_ref[...] = p_ref[0] + p_ref[1] + d_ref[...]

    return pl.pallas_call(
        body,
        grid=(nblk,),
        in_specs=[
            pl.BlockSpec((NC, BR, f), lambda i: (0, i, 0)),
            pl.BlockSpec((BR, f), lambda i: (i, 0)),
        ],
        out_specs=pl.BlockSpec((BR, f), lambda i: (i, 0)),
        out_shape=jax.ShapeDtypeStruct((n, f), jnp.float32),
    )(parts, dense)


def kernel(x, edge_index, edge_type, Wq, Wk, Wv, Wp):
    n, f = x.shape
    r_ = Wq.shape[0]
    e = edge_type.shape[0]
    assert e % NW == 0
    epw = e // NW
    assert epw % BATCH == 0 and f % L == 0 and n % BR == 0

    rn = r_ * n
    cnt_pad = ((2 * rn + NS * L - 1) // (NS * L)) * (NS * L)
    n_pad = ((n + NS * BATCH - 1) // (NS * BATCH)) * (NS * BATCH)
    assert r_ <= 4

    rowv = edge_index[0]
    colv = edge_index[1]
    typv = edge_type.astype(jnp.int32)

    cnts = _sc_counts(rowv, colv, typv,
                      n_nodes=n, n_rel=r_, epw=epw,
                      cnt_pad=cnt_pad).reshape(NC, cnt_pad)
    cd = cnts[:, :rn].reshape(NC * r_, n).T
    dd = cnts[:, rn:2 * rn].reshape(NC * r_, n).T

    s3, aux = _tc_reduce(x, cd, n_rel=r_, f=f)
    v3, g, dense = _tc_prepare(x, dd, s3, aux, Wq, Wk, Wv, Wp,
                               _interleave_perm(f), n_rel=r_, f=f)

    gtab = g.T.reshape(rn)
    parts = _sc_edge_pass(rowv, colv, typv,
                          v3.reshape(rn, f), gtab,
                          n_nodes=n, epw=epw, n_pad=n_pad, f=f)

    return _tc_combine(parts, dense, f=f)


# speedup vs baseline: 1.5485x; 1.5464x over previous
import functools

import jax
import jax.numpy as jnp
from jax import lax
from jax.experimental import pallas as pl
from jax.experimental.pallas import tpu as pltpu
from jax.experimental.pallas import tpu_sc as plsc

NC = 2
NS = 16
L = 16
NW = NC * NS

BATCH = 80
BR = 2000


def _sc_counts(rowv, colv, typv, *, n_nodes, n_rel, epw, cnt_pad):
    nb = epw // BATCH
    gpb = BATCH // L
    stripe = cnt_pad // NS
    assert nb % 2 == 1 and nb >= 3
    mesh = plsc.VectorSubcoreMesh(
        core_axis_name="c", subcore_axis_name="s",
        num_cores=NC, num_subcores=NS)

    @functools.partial(
        pl.kernel,
        out_type=jax.ShapeDtypeStruct((NC * cnt_pad,), jnp.float32),
        mesh=mesh,
        scratch_types=[
            pltpu.VMEM((epw,), jnp.int32),
            pltpu.VMEM((epw,), jnp.int32),
            pltpu.VMEM((epw,), jnp.int32),
            [pltpu.VMEM((BATCH,), jnp.int32) for _ in range(2)],
            [pltpu.VMEM((BATCH,), jnp.int32) for _ in range(2)],
            pltpu.VMEM((BATCH,), jnp.float32),
            pltpu.VMEM((stripe,), jnp.float32),
            pltpu.VMEM_SHARED((cnt_pad,), jnp.float32),
            [pltpu.SemaphoreType.DMA for _ in range(2)],
        ],
        compiler_params=pltpu.CompilerParams(use_tc_tiling_on_sc=False,
                                             needs_layout_passes=False),
    )
    def k(row_hbm, col_hbm, typ_hbm, out_hbm,
          rv, cv, tv, civs, divs, ones, zv, acc, sems):
        cid = lax.axis_index("c")
        sid = lax.axis_index("s")
        wid = sid * NC + cid
        base = wid * epw

        def zstep(i, carry):
            zv[pl.ds(i * L, L)] = jnp.zeros((L,), jnp.float32)
            return carry

        lax.fori_loop(0, stripe // L, zstep, 0)
        pltpu.sync_copy(zv, acc.at[pl.ds(sid * stripe, stripe)])
        for gq in range(gpb):
            ones[pl.ds(gq * L, L)] = jnp.ones((L,), jnp.float32)
        pltpu.sync_copy(row_hbm.at[pl.ds(base, epw)], rv)
        pltpu.sync_copy(col_hbm.at[pl.ds(base, epw)], cv)
        pltpu.sync_copy(typ_hbm.at[pl.ds(base, epw)], tv)
        plsc.subcore_barrier()

        def start(b, p):
            off = b * BATCH
            for gq in range(gpb):
                t = tv[pl.ds(off + gq * L, L)]
                cc = cv[pl.ds(off + gq * L, L)]
                rr = rv[pl.ds(off + gq * L, L)]
                civs[p][pl.ds(gq * L, L)] = t * n_nodes + cc
                divs[p][pl.ds(gq * L, L)] = t * n_nodes + rr + n_rel * n_nodes
            pltpu.async_copy(ones, acc.at[civs[p]], sems[p], add=True)
            pltpu.async_copy(ones, acc.at[divs[p]], sems[p], add=True)

        def finish(p):
            pltpu.make_async_copy(ones, acc.at[civs[p]], sems[p]).wait()
            pltpu.make_async_copy(ones, acc.at[divs[p]], sems[p]).wait()

        start(0, 0)

        def bstep(i, carry):
            b0 = 2 * i
            start(b0 + 1, 1)
            finish(0)
            start(b0 + 2, 0)
            finish(1)
            return carry

        lax.fori_loop(0, (nb - 1) // 2, bstep, 0)
        finish(0)
        plsc.subcore_barrier()
        pltpu.sync_copy(acc.at[pl.ds(sid * stripe, stripe)], zv)
        pltpu.sync_copy(zv,
                        out_hbm.at[pl.ds(cid * cnt_pad + sid * stripe, stripe)])

    return k(rowv, colv, typv)


def _sc_edge_pass(rowv, colv, typv, v3, gtab, *, n_nodes, epw, n_pad, f):
    chk = 2000
    bpc = chk // BATCH
    nb = epw // BATCH
    gpb = BATCH // L
    astripe = n_pad // NS
    rn = v3.shape[0]
    assert nb % 2 == 1 and nb >= 3
    g2 = jnp.broadcast_to(gtab.reshape(rn, 1), (rn, L))
    mesh = plsc.VectorSubcoreMesh(
        core_axis_name="c", subcore_axis_name="s",
        num_cores=NC, num_subcores=NS)

    @functools.partial(
        pl.kernel,
        out_type=jax.ShapeDtypeStruct((NC, n_pad, f), jnp.float32),
        mesh=mesh,
        scratch_types=[
            pltpu.VMEM((chk,), jnp.int32),
            pltpu.VMEM((chk,), jnp.int32),
            pltpu.VMEM((chk,), jnp.int32),
            [pltpu.VMEM((BATCH,), jnp.int32) for _ in range(2)],
            [pltpu.VMEM((BATCH,), jnp.int32) for _ in range(2)],
            [pltpu.VMEM((BATCH,), jnp.int32) for _ in range(2)],
            [pltpu.VMEM((BATCH, L), jnp.float32) for _ in range(2)],
            [pltpu.VMEM((BATCH, f), jnp.float32) for _ in range(2)],
            pltpu.VMEM_SHARED((n_pad, f), jnp.float32),
            [pltpu.SemaphoreType.DMA for _ in range(2)],
            [pltpu.SemaphoreType.DMA for _ in range(2)],
        ],
        compiler_params=pltpu.CompilerParams(use_tc_tiling_on_sc=False,
                                             needs_layout_passes=False),
    )
    def k(row_hbm, col_hbm, typ_hbm, v3_hbm, g_hbm, out_hbm,
          rv, cv, tv, srcvs, dstvs, widxs, wbs, rowss, acc,
          semvs, semws):
        cid = lax.axis_index("c")
        sid = lax.axis_index("s")
        wid = sid * NC + cid
        base = wid * epw
        rows0 = rowss[0]

        def zrow(j, carry):
            for kq in range(f // L):
                rows0[j, pl.ds(kq * L, L)] = jnp.zeros((L,), jnp.float32)
            return carry

        lax.fori_loop(0, BATCH, zrow, 0)
        for ch in range(astripe // BATCH):
            pltpu.sync_copy(
                rows0, acc.at[pl.ds(sid * astripe + ch * BATCH, BATCH)])
        plsc.subcore_barrier()

        def start(b, p):
            @pl.when(lax.rem(b, bpc) == 0)
            def _():
                coff = base + (b // bpc) * chk
                pltpu.sync_copy(row_hbm.at[pl.ds(coff, chk)], rv)
                pltpu.sync_copy(col_hbm.at[pl.ds(coff, chk)], cv)
                pltpu.sync_copy(typ_hbm.at[pl.ds(coff, chk)], tv)
            off = lax.rem(b, bpc) * BATCH
            for gq in range(gpb):
                t = tv[pl.ds(off + gq * L, L)]
                cc = cv[pl.ds(off + gq * L, L)]
                rr = rv[pl.ds(off + gq * L, L)]
                srcvs[p][pl.ds(gq * L, L)] = t * n_nodes + cc
                dstvs[p][pl.ds(gq * L, L)] = rr
                widxs[p][pl.ds(gq * L, L)] = t * n_nodes + rr
            pltpu.async_copy(v3_hbm.at[srcvs[p]], rowss[p], semvs[p])
            pltpu.async_copy(g_hbm.at[widxs[p]], wbs[p], semws[p])

        def finish(p):
            pltpu.make_async_copy(v3_hbm.at[srcvs[p]], rowss[p],
                                  semvs[p]).wait()
            pltpu.make_async_copy(g_hbm.at[widxs[p]], wbs[p],
                                  semws[p]).wait()
            rows = rowss[p]
            wb = wbs[p]

            def escale(jj, c2):
                for u in range(4):
                    j = jj * 4 + u
                    wrep = wb[j, pl.ds(0, L)]
                    for kq in range(f // L):
                        rows[j, pl.ds(kq * L, L)] = (
                            rows[j, pl.ds(kq * L, L)] * wrep)
                return c2

            lax.fori_loop(0, BATCH // 4, escale, 0)
            pltpu.sync_copy(rows, acc.at[dstvs[p]], add=True)

        start(0, 0)

        def bstep(i, carry):
            b0 = 2 * i
            start(b0 + 1, 1)
            finish(0)
            start(b0 + 2, 0)
            finish(1)
            return carry

        lax.fori_loop(0, (nb - 1) // 2, bstep, 0)
        finish(0)
        plsc.subcore_barrier()
        for ch in range(astripe // BATCH):
            off2 = sid * astripe + ch * BATCH
            pltpu.sync_copy(acc.at[pl.ds(off2, BATCH)], rows0)
            pltpu.sync_copy(rows0, out_hbm.at[cid, pl.ds(off2, BATCH)])

    return k(rowv, colv, typv, v3, g2)


def _tc_reduce(x, cd, *, n_rel, f):
    n = x.shape[0]
    nblk = n // BR

    def body(xb, cb, s_ref, aux_ref):
        pid = pl.program_id(0)

        @pl.when(pid == 0)
        def _():
            s_ref[...] = jnp.zeros_like(s_ref)
            aux_ref[...] = jnp.zeros_like(aux_ref)

        xv = xb[...]
        cv = cb[...]
        ones_row = jnp.ones((1, f), jnp.float32)
        for r in range(n_rel):
            cr = cv[:, r:r + 1] + cv[:, n_rel + r:n_rel + r + 1]
            cr128 = lax.dot_general(cr, ones_row, (((1,), (0,)), ((), ())),
                                    preferred_element_type=jnp.float32)
            xc = xv * cr128
            s_ref[r] += lax.dot_general(
                xc, xv, (((0,), (0,)), ((), ())),
                preferred_element_type=jnp.float32)
            s_r = jnp.sum(xc, axis=0, keepdims=True)
            cnt_row = jnp.sum(cr128, axis=0, keepdims=True)
            upd = jnp.concatenate(
                [s_r, cnt_row, jnp.zeros((6, f), jnp.float32)], axis=0)
            aux_ref[r] += upd

    return pl.pallas_call(
        body,
        grid=(nblk,),
        in_specs=[
            pl.BlockSpec((BR, f), lambda i: (i, 0)),
            pl.BlockSpec((BR, NC * n_rel), lambda i: (i, 0)),
        ],
        out_specs=[
            pl.BlockSpec((n_rel, f, f), lambda i: (0, 0, 0)),
            pl.BlockSpec((n_rel, 8, f), lambda i: (0, 0, 0)),
        ],
        out_shape=[
            jax.ShapeDtypeStruct((n_rel, f, f), jnp.float32),
            jax.ShapeDtypeStruct((n_rel, 8, f), jnp.float32),
        ],
    )(x, cd)


def _tc_prepare(x, dd, s3, aux, wq, wk, wv, wp, *, n_rel, f):
    n = x.shape[0]
    nblk = n // BR

    def body(xb, db, s_ref, aux_ref, wq_ref, wk_ref, wv_ref, wp_ref,
             v3_ref, g_ref, dense_ref):
        xv = xb[...]
        dv = db[...]
        wp_ = wp_ref[...]
        ones_row = jnp.ones((1, f), jnp.float32)
        dense = jnp.zeros((BR, f), jnp.float32)
        for r in range(n_rel):
            wqr = wq_ref[r]
            wkr = wk_ref[r]
            wvr = wv_ref[r]
            s_mat = s_ref[r]
            sv = lax.dot_general(s_mat, wvr, (((1,), (0,)), ((), ())),
                                 preferred_element_type=jnp.float32)
            kvs = lax.dot_general(wkr, sv, (((0,), (0,)), ((), ())),
                                  preferred_element_type=jnp.float32)
            a2 = wqr @ kvs @ wp_
            s_row = aux_ref[r, 0:1, :]
            ks_row = lax.dot_general(s_row, wkr, (((1,), (0,)), ((), ())),
                                     preferred_element_type=jnp.float32)
            b_row = lax.dot_general(ks_row, wqr, (((1,), (1,)), ((), ())),
                                    preferred_element_type=jnp.float32)
            b_mat = lax.dot_general(b_row, ones_row, (((0,), (0,)), ((), ())),
                                    preferred_element_type=jnp.float32)
            cnt_row = aux_ref[r, 1:2, :]
            den = (xv @ b_mat) + cnt_row
            den = jnp.where(den == 0.0, 1.0, den)
            g_val = cnt_row / den
            g_ref[:, r:r + 1] = g_val[:, 0:1]
            v3_ref[r] = xv @ (wvr @ wp_)
            dr = dv[:, r:r + 1] + dv[:, n_rel + r:n_rel + r + 1]
            dr128 = lax.dot_general(dr, ones_row, (((1,), (0,)), ((), ())),
                                    preferred_element_type=jnp.float32)
            dense = dense + (xv @ a2) * (dr128 / den)
        dense_ref[...] = dense

    return pl.pallas_call(
        body,
        grid=(nblk,),
        in_specs=[
            pl.BlockSpec((BR, f), lambda i: (i, 0)),
            pl.BlockSpec((BR, NC * n_rel), lambda i: (i, 0)),
            pl.BlockSpec((n_rel, f, f), lambda i: (0, 0, 0)),
            pl.BlockSpec((n_rel, 8, f), lambda i: (0, 0, 0)),
            pl.BlockSpec((n_rel, f, f), lambda i: (0, 0, 0)),
            pl.BlockSpec((n_rel, f, f), lambda i: (0, 0, 0)),
            pl.BlockSpec((n_rel, f, f), lambda i: (0, 0, 0)),
            pl.BlockSpec((f, f), lambda i: (0, 0)),
        ],
        out_specs=[
            pl.BlockSpec((n_rel, BR, f), lambda i: (0, i, 0)),
            pl.BlockSpec((BR, n_rel), lambda i: (i, 0)),
            pl.BlockSpec((BR, f), lambda i: (i, 0)),
        ],
        out_shape=[
            jax.ShapeDtypeStruct((n_rel, n, f), jnp.float32),
            jax.ShapeDtypeStruct((n, n_rel), jnp.float32),
            jax.ShapeDtypeStruct((n, f), jnp.float32),
        ],
    )(x, dd, s3, aux, wq, wk, wv, wp)


def _tc_combine(parts, dense, *, f):
    n = dense.shape[0]
    nblk = n // BR

    def body(p_ref, d_ref, o_ref):
        o_ref[...] = p_ref[0] + p_ref[1] + d_ref[...]

    return pl.pallas_call(
        body,
        grid=(nblk,),
        in_specs=[
            pl.BlockSpec((NC, BR, f), lambda i: (0, i, 0)),
            pl.BlockSpec((BR, f), lambda i: (i, 0)),
        ],
        out_specs=pl.BlockSpec((BR, f), lambda i: (i, 0)),
        out_shape=jax.ShapeDtypeStruct((n, f), jnp.float32),
    )(parts, dense)


def kernel(x, edge_index, edge_type, Wq, Wk, Wv, Wp):
    n, f = x.shape
    r_ = Wq.shape[0]
    e = edge_type.shape[0]
    assert e % NW == 0
    epw = e // NW
    assert epw % BATCH == 0 and f % L == 0 and n % BR == 0

    rn = r_ * n
    cnt_pad = ((2 * rn + NS * L - 1) // (NS * L)) * (NS * L)
    n_pad = ((n + NS * BATCH - 1) // (NS * BATCH)) * (NS * BATCH)
    assert r_ <= 4

    rowv = edge_index[0]
    colv = edge_index[1]
    typv = edge_type.astype(jnp.int32)

    cnts = _sc_counts(rowv, colv, typv,
                      n_nodes=n, n_rel=r_, epw=epw,
                      cnt_pad=cnt_pad).reshape(NC, cnt_pad)
    cd = cnts[:, :rn].reshape(NC * r_, n).T
    dd = cnts[:, rn:2 * rn].reshape(NC * r_, n).T

    s3, aux = _tc_reduce(x, cd, n_rel=r_, f=f)
    v3, g, dense = _tc_prepare(x, dd, s3, aux, Wq, Wk, Wv, Wp, n_rel=r_, f=f)

    gtab = g.T.reshape(rn)
    parts = _sc_edge_pass(rowv, colv, typv,
                          v3.reshape(rn, f), gtab,
                          n_nodes=n, epw=epw, n_pad=n_pad, f=f)

    return _tc_combine(parts, dense, f=f)


# speedup vs baseline: 1.6408x; 1.0596x over previous
import functools

import jax
import jax.numpy as jnp
from jax import lax
from jax.experimental import pallas as pl
from jax.experimental.pallas import tpu as pltpu
from jax.experimental.pallas import tpu_sc as plsc

NC = 2
NS = 16
L = 16
NW = NC * NS

BATCH = 80
BR = 2000


def _sc_counts(rowv, colv, typv, *, n_nodes, n_rel, epw, cnt_pad):
    nb = epw // BATCH
    gpb = BATCH // L
    stripe = cnt_pad // NS
    assert nb % 2 == 1 and nb >= 3
    mesh = plsc.VectorSubcoreMesh(
        core_axis_name="c", subcore_axis_name="s",
        num_cores=NC, num_subcores=NS)

    @functools.partial(
        pl.kernel,
        out_type=jax.ShapeDtypeStruct((NC * cnt_pad,), jnp.float32),
        mesh=mesh,
        scratch_types=[
            pltpu.VMEM((epw,), jnp.int32),
            pltpu.VMEM((epw,), jnp.int32),
            pltpu.VMEM((epw,), jnp.int32),
            [pltpu.VMEM((BATCH,), jnp.int32) for _ in range(2)],
            [pltpu.VMEM((BATCH,), jnp.int32) for _ in range(2)],
            pltpu.VMEM((BATCH,), jnp.float32),
            pltpu.VMEM((stripe,), jnp.float32),
            pltpu.VMEM_SHARED((cnt_pad,), jnp.float32),
            [pltpu.SemaphoreType.DMA for _ in range(2)],
        ],
        compiler_params=pltpu.CompilerParams(use_tc_tiling_on_sc=False,
                                             needs_layout_passes=False),
    )
    def k(row_hbm, col_hbm, typ_hbm, out_hbm,
          rv, cv, tv, civs, divs, ones, zv, acc, sems):
        cid = lax.axis_index("c")
        sid = lax.axis_index("s")
        wid = sid * NC + cid
        base = wid * epw

        def zstep(i, carry):
            zv[pl.ds(i * L, L)] = jnp.zeros((L,), jnp.float32)
            return carry

        lax.fori_loop(0, stripe // L, zstep, 0)
        pltpu.sync_copy(zv, acc.at[pl.ds(sid * stripe, stripe)])
        for gq in range(gpb):
            ones[pl.ds(gq * L, L)] = jnp.ones((L,), jnp.float32)
        pltpu.sync_copy(row_hbm.at[pl.ds(base, epw)], rv)
        pltpu.sync_copy(col_hbm.at[pl.ds(base, epw)], cv)
        pltpu.sync_copy(typ_hbm.at[pl.ds(base, epw)], tv)
        plsc.subcore_barrier()

        def start(b, p):
            off = b * BATCH
            for gq in range(gpb):
                t = tv[pl.ds(off + gq * L, L)]
                cc = cv[pl.ds(off + gq * L, L)]
                rr = rv[pl.ds(off + gq * L, L)]
                civs[p][pl.ds(gq * L, L)] = t * n_nodes + cc
                divs[p][pl.ds(gq * L, L)] = t * n_nodes + rr + n_rel * n_nodes
            pltpu.async_copy(ones, acc.at[civs[p]], sems[p], add=True)
            pltpu.async_copy(ones, acc.at[divs[p]], sems[p], add=True)

        def finish(p):
            pltpu.make_async_copy(ones, acc.at[civs[p]], sems[p]).wait()
            pltpu.make_async_copy(ones, acc.at[divs[p]], sems[p]).wait()

        start(0, 0)

        def bstep(i, carry):
            b0 = 2 * i
            start(b0 + 1, 1)
            finish(0)
            start(b0 + 2, 0)
            finish(1)
            return carry

        lax.fori_loop(0, (nb - 1) // 2, bstep, 0)
        finish(0)
        plsc.subcore_barrier()
        pltpu.sync_copy(acc.at[pl.ds(sid * stripe, stripe)], zv)
        pltpu.sync_copy(zv,
                        out_hbm.at[pl.ds(cid * cnt_pad + sid * stripe, stripe)])

    return k(rowv, colv, typv)


def _sc_edge_pass(rowv, colv, typv, v3, gtab, *, n_nodes, epw, n_pad, f):
    chk = 2000
    bpc = chk // BATCH
    nb = epw // BATCH
    gpb = BATCH // L
    astripe = n_pad // NS
    rn = v3.shape[0]
    assert nb % 2 == 1 and nb >= 3
    mesh = plsc.VectorSubcoreMesh(
        core_axis_name="c", subcore_axis_name="s",
        num_cores=NC, num_subcores=NS)

    @functools.partial(
        pl.kernel,
        out_type=jax.ShapeDtypeStruct((NC, n_pad, f), jnp.float32),
        mesh=mesh,
        scratch_types=[
            pltpu.VMEM((chk,), jnp.int32),
            pltpu.VMEM((chk,), jnp.int32),
            pltpu.VMEM((chk,), jnp.int32),
            [pltpu.VMEM((BATCH,), jnp.int32) for _ in range(2)],
            [pltpu.VMEM((BATCH,), jnp.int32) for _ in range(2)],
            [pltpu.VMEM((BATCH,), jnp.int32) for _ in range(2)],
            [pltpu.VMEM((BATCH, L), jnp.float32) for _ in range(2)],
            [pltpu.VMEM((BATCH, f), jnp.float32) for _ in range(2)],
            pltpu.VMEM_SHARED((n_pad, f), jnp.float32),
            [pltpu.SemaphoreType.DMA for _ in range(2)],
            [pltpu.SemaphoreType.DMA for _ in range(2)],
            [pltpu.SemaphoreType.DMA for _ in range(2)],
        ],
        compiler_params=pltpu.CompilerParams(use_tc_tiling_on_sc=False,
                                             needs_layout_passes=False),
    )
    def k(row_hbm, col_hbm, typ_hbm, v3_hbm, g_hbm, out_hbm,
          rv, cv, tv, srcvs, dstvs, widxs, wbs, rowss, acc,
          semvs, semws, semss):
        cid = lax.axis_index("c")
        sid = lax.axis_index("s")
        wid = sid * NC + cid
        base = wid * epw
        rows0 = rowss[0]

        def zrow(j, carry):
            for kq in range(f // L):
                rows0[j, pl.ds(kq * L, L)] = jnp.zeros((L,), jnp.float32)
            return carry

        lax.fori_loop(0, BATCH, zrow, 0)
        for ch in range(astripe // BATCH):
            pltpu.sync_copy(
                rows0, acc.at[pl.ds(sid * astripe + ch * BATCH, BATCH)])
        plsc.subcore_barrier()

        def start(b, p):
            @pl.when(b >= 2)
            def _():
                pltpu.make_async_copy(rowss[p], acc.at[dstvs[p]],
                                      semss[p]).wait()

            @pl.when(lax.rem(b, bpc) == 0)
            def _():
                coff = base + (b // bpc) * chk
                pltpu.sync_copy(row_hbm.at[pl.ds(coff, chk)], rv)
                pltpu.sync_copy(col_hbm.at[pl.ds(coff, chk)], cv)
                pltpu.sync_copy(typ_hbm.at[pl.ds(coff, chk)], tv)
            off = lax.rem(b, bpc) * BATCH
            for gq in range(gpb):
                t = tv[pl.ds(off + gq * L, L)]
                cc = cv[pl.ds(off + gq * L, L)]
                rr = rv[pl.ds(off + gq * L, L)]
                srcvs[p][pl.ds(gq * L, L)] = t * n_nodes + cc
                dstvs[p][pl.ds(gq * L, L)] = rr
                widxs[p][pl.ds(gq * L, L)] = t * n_nodes + rr
            pltpu.async_copy(v3_hbm.at[srcvs[p]], rowss[p], semvs[p])
            pltpu.async_copy(g_hbm.at[widxs[p]], wbs[p], semws[p])

        def finish(p):
            pltpu.make_async_copy(v3_hbm.at[srcvs[p]], rowss[p],
                                  semvs[p]).wait()
            pltpu.make_async_copy(g_hbm.at[widxs[p]], wbs[p],
                                  semws[p]).wait()
            rows = rowss[p]
            wb = wbs[p]

            def escale(jj, c2):
                for u in range(8):
                    j = jj * 8 + u
                    wrep = wb[j, pl.ds(0, L)]
                    for kq in range(f // L):
                        rows[j, pl.ds(kq * L, L)] = (
                            rows[j, pl.ds(kq * L, L)] * wrep)
                return c2

            lax.fori_loop(0, BATCH // 8, escale, 0)
            pltpu.async_copy(rows, acc.at[dstvs[p]], semss[p], add=True)

        start(0, 0)

        def bstep(i, carry):
            b0 = 2 * i
            start(b0 + 1, 1)
            finish(0)
            start(b0 + 2, 0)
            finish(1)
            return carry

        lax.fori_loop(0, (nb - 1) // 2, bstep, 0)
        finish(0)
        for p in range(2):
            pltpu.make_async_copy(rowss[p], acc.at[dstvs[p]],
                                  semss[p]).wait()
        plsc.subcore_barrier()
        for ch in range(astripe // BATCH):
            off2 = sid * astripe + ch * BATCH
            pltpu.sync_copy(acc.at[pl.ds(off2, BATCH)], rows0)
            pltpu.sync_copy(rows0, out_hbm.at[cid, pl.ds(off2, BATCH)])

    return k(rowv, colv, typv, v3, gtab)


def _tc_reduce(x, cd, *, n_rel, f):
    n = x.shape[0]
    nblk = n // BR

    def body(xb, cb, s_ref, aux_ref):
        pid = pl.program_id(0)

        @pl.when(pid == 0)
        def _():
            s_ref[...] = jnp.zeros_like(s_ref)
            aux_ref[...] = jnp.zeros_like(aux_ref)

        xv = xb[...]
        cv = cb[...]
        ones_row = jnp.ones((1, f), jnp.float32)
        for r in range(n_rel):
            cr = cv[:, r:r + 1] + cv[:, n_rel + r:n_rel + r + 1]
            cr128 = lax.dot_general(cr, ones_row, (((1,), (0,)), ((), ())),
                                    preferred_element_type=jnp.float32)
            xc = xv * cr128
            s_ref[r] += lax.dot_general(
                xc, xv, (((0,), (0,)), ((), ())),
                preferred_element_type=jnp.float32)
            s_r = jnp.sum(xc, axis=0, keepdims=True)
            cnt_row = jnp.sum(cr128, axis=0, keepdims=True)
            upd = jnp.concatenate(
                [s_r, cnt_row, jnp.zeros((6, f), jnp.float32)], axis=0)
            aux_ref[r] += upd

    return pl.pallas_call(
        body,
        grid=(nblk,),
        in_specs=[
            pl.BlockSpec((BR, f), lambda i: (i, 0)),
            pl.BlockSpec((BR, NC * n_rel), lambda i: (i, 0)),
        ],
        out_specs=[
            pl.BlockSpec((n_rel, f, f), lambda i: (0, 0, 0)),
            pl.BlockSpec((n_rel, 8, f), lambda i: (0, 0, 0)),
        ],
        out_shape=[
            jax.ShapeDtypeStruct((n_rel, f, f), jnp.float32),
            jax.ShapeDtypeStruct((n_rel, 8, f), jnp.float32),
        ],
    )(x, cd)


def _tc_prepare(x, dd, s3, aux, wq, wk, wv, wp, *, n_rel, f):
    n = x.shape[0]
    nblk = n // BR

    def body(xb, db, s_ref, aux_ref, wq_ref, wk_ref, wv_ref, wp_ref,
             v3_ref, g_ref, dense_ref):
        xv = xb[...]
        dv = db[...]
        wp_ = wp_ref[...]
        ones_row = jnp.ones((1, f), jnp.float32)
        dense = jnp.zeros((BR, f), jnp.float32)
        for r in range(n_rel):
            wqr = wq_ref[r]
            wkr = wk_ref[r]
            wvr = wv_ref[r]
            s_mat = s_ref[r]
            sv = lax.dot_general(s_mat, wvr, (((1,), (0,)), ((), ())),
                                 preferred_element_type=jnp.float32)
            kvs = lax.dot_general(wkr, sv, (((0,), (0,)), ((), ())),
                                  preferred_element_type=jnp.float32)
            a2 = wqr @ kvs @ wp_
            s_row = aux_ref[r, 0:1, :]
            ks_row = lax.dot_general(s_row, wkr, (((1,), (0,)), ((), ())),
                                     preferred_element_type=jnp.float32)
            b_row = lax.dot_general(ks_row, wqr, (((1,), (1,)), ((), ())),
                                    preferred_element_type=jnp.float32)
            b_mat = lax.dot_general(b_row, ones_row, (((0,), (0,)), ((), ())),
                                    preferred_element_type=jnp.float32)
            cnt_row = aux_ref[r, 1:2, :]
            den = (xv @ b_mat) + cnt_row
            den = jnp.where(den == 0.0, 1.0, den)
            g_val = cnt_row / den
            g_ref[r] = g_val[:, 0:L]
            v3_ref[r] = xv @ (wvr @ wp_)
            dr = dv[:, r:r + 1] + dv[:, n_rel + r:n_rel + r + 1]
            dr128 = lax.dot_general(dr, ones_row, (((1,), (0,)), ((), ())),
                                    preferred_element_type=jnp.float32)
            dense = dense + (xv @ a2) * (dr128 / den)
        dense_ref[...] = dense

    return pl.pallas_call(
        body,
        grid=(nblk,),
        in_specs=[
            pl.BlockSpec((BR, f), lambda i: (i, 0)),
            pl.BlockSpec((BR, NC * n_rel), lambda i: (i, 0)),
            pl.BlockSpec((n_rel, f, f), lambda i: (0, 0, 0)),
            pl.BlockSpec((n_rel, 8, f), lambda i: (0, 0, 0)),
            pl.BlockSpec((n_rel, f, f), lambda i: (0, 0, 0)),
            pl.BlockSpec((n_rel, f, f), lambda i: (0, 0, 0)),
            pl.BlockSpec((n_rel, f, f), lambda i: (0, 0, 0)),
            pl.BlockSpec((f, f), lambda i: (0, 0)),
        ],
        out_specs=[
            pl.BlockSpec((n_rel, BR, f), lambda i: (0, i, 0)),
            pl.BlockSpec((n_rel, BR, L), lambda i: (0, i, 0)),
            pl.BlockSpec((BR, f), lambda i: (i, 0)),
        ],
        out_shape=[
            jax.ShapeDtypeStruct((n_rel, n, f), jnp.float32),
            jax.ShapeDtypeStruct((n_rel, n, L), jnp.float32),
            jax.ShapeDtypeStruct((n, f), jnp.float32),
        ],
    )(x, dd, s3, aux, wq, wk, wv, wp)


def _tc_combine(parts, dense, *, f):
    n = dense.shape[0]
    nblk = n // BR

    def body(p_ref, d_ref, o_ref):
        o_ref[...] = p_ref[0] + p_ref[1] + d_ref[...]

    return pl.pallas_call(
        body,
        grid=(nblk,),
        in_specs=[
            pl.BlockSpec((NC, BR, f), lambda i: (0, i, 0)),
            pl.BlockSpec((BR, f), lambda i: (i, 0)),
        ],
        out_specs=pl.BlockSpec((BR, f), lambda i: (i, 0)),
        out_shape=jax.ShapeDtypeStruct((n, f), jnp.float32),
    )(parts, dense)


def kernel(x, edge_index, edge_type, Wq, Wk, Wv, Wp):
    n, f = x.shape
    r_ = Wq.shape[0]
    e = edge_type.shape[0]
    assert e % NW == 0
    epw = e // NW
    assert epw % BATCH == 0 and f % L == 0 and n % BR == 0

    rn = r_ * n
    cnt_pad = ((2 * rn + NS * L - 1) // (NS * L)) * (NS * L)
    n_pad = ((n + NS * BATCH - 1) // (NS * BATCH)) * (NS * BATCH)
    assert r_ <= 4

    rowv = edge_index[0]
    colv = edge_index[1]
    typv = edge_type.astype(jnp.int32)

    cnts = _sc_counts(rowv, colv, typv,
                      n_nodes=n, n_rel=r_, epw=epw,
                      cnt_pad=cnt_pad).reshape(NC, cnt_pad)
    cd = cnts[:, :rn].reshape(NC * r_, n).T
    dd = cnts[:, rn:2 * rn].reshape(NC * r_, n).T

    s3, aux = _tc_reduce(x, cd, n_rel=r_, f=f)
    v3, g, dense = _tc_prepare(x, dd, s3, aux, Wq, Wk, Wv, Wp, n_rel=r_, f=f)

    parts = _sc_edge_pass(rowv, colv, typv,
                          v3.reshape(rn, f), g.reshape(rn, L),
                          n_nodes=n, epw=epw, n_pad=n_pad, f=f)

    return _tc_combine(parts, dense, f=f)


# speedup vs baseline: 1.6465x; 1.0035x over previous
import functools

import jax
import jax.numpy as jnp
from jax import lax
from jax.experimental import pallas as pl
from jax.experimental.pallas import tpu as pltpu
from jax.experimental.pallas import tpu_sc as plsc

NC = 2
NS = 16
L = 16
NW = NC * NS

BATCH = 80
BR = 2000


def _sc_counts(rowv, colv, typv, *, n_nodes, n_rel, epw, cnt_pad):
    nb = epw // BATCH
    gpb = BATCH // L
    stripe = cnt_pad // NS
    assert nb % 2 == 1 and nb >= 3
    mesh = plsc.VectorSubcoreMesh(
        core_axis_name="c", subcore_axis_name="s",
        num_cores=NC, num_subcores=NS)

    @functools.partial(
        pl.kernel,
        out_type=jax.ShapeDtypeStruct((NC * cnt_pad,), jnp.float32),
        mesh=mesh,
        scratch_types=[
            pltpu.VMEM((epw,), jnp.int32),
            pltpu.VMEM((epw,), jnp.int32),
            pltpu.VMEM((epw,), jnp.int32),
            [pltpu.VMEM((BATCH,), jnp.int32) for _ in range(2)],
            [pltpu.VMEM((BATCH,), jnp.int32) for _ in range(2)],
            pltpu.VMEM((BATCH,), jnp.float32),
            pltpu.VMEM((stripe,), jnp.float32),
            pltpu.VMEM_SHARED((cnt_pad,), jnp.float32),
            [pltpu.SemaphoreType.DMA for _ in range(2)],
        ],
        compiler_params=pltpu.CompilerParams(use_tc_tiling_on_sc=False,
                                             needs_layout_passes=False),
    )
    def k(row_hbm, col_hbm, typ_hbm, out_hbm,
          rv, cv, tv, civs, divs, ones, zv, acc, sems):
        cid = lax.axis_index("c")
        sid = lax.axis_index("s")
        wid = sid * NC + cid
        base = wid * epw

        def zstep(i, carry):
            zv[pl.ds(i * L, L)] = jnp.zeros((L,), jnp.float32)
            return carry

        lax.fori_loop(0, stripe // L, zstep, 0)
        pltpu.sync_copy(zv, acc.at[pl.ds(sid * stripe, stripe)])
        for gq in range(gpb):
            ones[pl.ds(gq * L, L)] = jnp.ones((L,), jnp.float32)
        pltpu.sync_copy(row_hbm.at[pl.ds(base, epw)], rv)
        pltpu.sync_copy(col_hbm.at[pl.ds(base, epw)], cv)
        pltpu.sync_copy(typ_hbm.at[pl.ds(base, epw)], tv)
        plsc.subcore_barrier()

        def start(b, p):
            off = b * BATCH
            for gq in range(gpb):
                t = tv[pl.ds(off + gq * L, L)]
                cc = cv[pl.ds(off + gq * L, L)]
                rr = rv[pl.ds(off + gq * L, L)]
                civs[p][pl.ds(gq * L, L)] = t * n_nodes + cc
                divs[p][pl.ds(gq * L, L)] = t * n_nodes + rr + n_rel * n_nodes
            pltpu.async_copy(ones, acc.at[civs[p]], sems[p], add=True)
            pltpu.async_copy(ones, acc.at[divs[p]], sems[p], add=True)

        def finish(p):
            pltpu.make_async_copy(ones, acc.at[civs[p]], sems[p]).wait()
            pltpu.make_async_copy(ones, acc.at[divs[p]], sems[p]).wait()

        start(0, 0)

        def bstep(i, carry):
            b0 = 2 * i
            start(b0 + 1, 1)
            finish(0)
            start(b0 + 2, 0)
            finish(1)
            return carry

        lax.fori_loop(0, (nb - 1) // 2, bstep, 0)
        finish(0)
        plsc.subcore_barrier()
        pltpu.sync_copy(acc.at[pl.ds(sid * stripe, stripe)], zv)
        pltpu.sync_copy(zv,
                        out_hbm.at[pl.ds(cid * cnt_pad + sid * stripe, stripe)])

    return k(rowv, colv, typv)


def _sc_edge_pass(rowv, colv, typv, v3, gtab, *, n_nodes, epw, n_pad, f):
    chk = 2000
    bpc = chk // BATCH
    nb = epw // BATCH
    gpb = BATCH // L
    astripe = n_pad // NS
    rn = v3.shape[0]
    assert nb % 2 == 1 and nb >= 3
    mesh = plsc.VectorSubcoreMesh(
        core_axis_name="c", subcore_axis_name="s",
        num_cores=NC, num_subcores=NS)

    @functools.partial(
        pl.kernel,
        out_type=jax.ShapeDtypeStruct((NC, n_pad, f), jnp.float32),
        mesh=mesh,
        scratch_types=[
            pltpu.VMEM((chk,), jnp.int32),
            pltpu.VMEM((chk,), jnp.int32),
            pltpu.VMEM((chk,), jnp.int32),
            [pltpu.VMEM((BATCH,), jnp.int32) for _ in range(2)],
            [pltpu.VMEM((BATCH,), jnp.int32) for _ in range(2)],
            [pltpu.VMEM((BATCH,), jnp.int32) for _ in range(2)],
            [pltpu.VMEM((BATCH, L), jnp.float32) for _ in range(2)],
            [pltpu.VMEM((BATCH, f), jnp.float32) for _ in range(2)],
            pltpu.VMEM_SHARED((n_pad, f), jnp.float32),
            [pltpu.SemaphoreType.DMA for _ in range(2)],
            [pltpu.SemaphoreType.DMA for _ in range(2)],
            [pltpu.SemaphoreType.DMA for _ in range(2)],
        ],
        compiler_params=pltpu.CompilerParams(use_tc_tiling_on_sc=False,
                                             needs_layout_passes=False),
    )
    def k(row_hbm, col_hbm, typ_hbm, v3_hbm, g_hbm, out_hbm,
          rv, cv, tv, srcvs, dstvs, widxs, wbs, rowss, acc,
          semvs, semws, semss):
        cid = lax.axis_index("c")
        sid = lax.axis_index("s")
        wid = sid * NC + cid
        base = wid * epw
        rows0 = rowss[0]

        def zrow(j, carry):
            for kq in range(f // L):
                rows0[j, pl.ds(kq * L, L)] = jnp.zeros((L,), jnp.float32)
            return carry

        lax.fori_loop(0, BATCH, zrow, 0)
        for ch in range(astripe // BATCH):
            pltpu.sync_copy(
                rows0, acc.at[pl.ds(sid * astripe + ch * BATCH, BATCH)])
        plsc.subcore_barrier()

        def start(b, p):
            @pl.when(b >= 2)
            def _():
                pltpu.make_async_copy(rowss[p], acc.at[dstvs[p]],
                                      semss[p]).wait()

            @pl.when(lax.rem(b, bpc) == 0)
            def _():
                coff = base + (b // bpc) * chk
                pltpu.sync_copy(row_hbm.at[pl.ds(coff, chk)], rv)
                pltpu.sync_copy(col_hbm.at[pl.ds(coff, chk)], cv)
                pltpu.sync_copy(typ_hbm.at[pl.ds(coff, chk)], tv)
            off = lax.rem(b, bpc) * BATCH
            for gq in range(gpb):
                t = tv[pl.ds(off + gq * L, L)]
                cc = cv[pl.ds(off + gq * L, L)]
                rr = rv[pl.ds(off + gq * L, L)]
                srcvs[p][pl.ds(gq * L, L)] = t * n_nodes + cc
                dstvs[p][pl.ds(gq * L, L)] = rr
                widxs[p][pl.ds(gq * L, L)] = t * n_nodes + rr
            pltpu.async_copy(v3_hbm.at[srcvs[p]], rowss[p], semvs[p])
            pltpu.async_copy(g_hbm.at[widxs[p]], wbs[p], semws[p])

        def finish(p):
            pltpu.make_async_copy(v3_hbm.at[srcvs[p]], rowss[p],
                                  semvs[p]).wait()
            pltpu.make_async_copy(g_hbm.at[widxs[p]], wbs[p],
                                  semws[p]).wait()
            rows = rowss[p]
            wb = wbs[p]

            def escale(jj, c2):
                for u in range(8):
                    j = jj * 8 + u
                    wrep = wb[j, pl.ds(0, L)]
                    for kq in range(f // L):
                        rows[j, pl.ds(kq * L, L)] = (
                            rows[j, pl.ds(kq * L, L)] * wrep)
                return c2

            lax.fori_loop(0, BATCH // 8, escale, 0)
            pltpu.async_copy(rows, acc.at[dstvs[p]], semss[p], add=True)

        start(0, 0)

        def bstep(i, carry):
            b0 = 2 * i
            start(b0 + 1, 1)
            finish(0)
            start(b0 + 2, 0)
            finish(1)
            return carry

        lax.fori_loop(0, (nb - 1) // 2, bstep, 0)
        finish(0)
        for p in range(2):
            pltpu.make_async_copy(rowss[p], acc.at[dstvs[p]],
                                  semss[p]).wait()
        plsc.subcore_barrier()
        pltpu.sync_copy(acc.at[pl.ds(sid * astripe, astripe)],
                        out_hbm.at[cid, pl.ds(sid * astripe, astripe)])

    return k(rowv, colv, typv, v3, gtab)


def _tc_reduce(x, cd, *, n_rel, f):
    n = x.shape[0]
    nblk = n // BR

    def body(xb, cb, s_ref, aux_ref):
        pid = pl.program_id(0)

        @pl.when(pid == 0)
        def _():
            s_ref[...] = jnp.zeros_like(s_ref)
            aux_ref[...] = jnp.zeros_like(aux_ref)

        xv = xb[...]
        cv = cb[...]
        ones_row = jnp.ones((1, f), jnp.float32)
        for r in range(n_rel):
            cr = cv[:, r:r + 1] + cv[:, n_rel + r:n_rel + r + 1]
            cr128 = lax.dot_general(cr, ones_row, (((1,), (0,)), ((), ())),
                                    preferred_element_type=jnp.float32)
            xc = xv * cr128
            s_ref[r] += lax.dot_general(
                xc, xv, (((0,), (0,)), ((), ())),
                preferred_element_type=jnp.float32)
            s_r = jnp.sum(xc, axis=0, keepdims=True)
            cnt_row = jnp.sum(cr128, axis=0, keepdims=True)
            upd = jnp.concatenate(
                [s_r, cnt_row, jnp.zeros((6, f), jnp.float32)], axis=0)
            aux_ref[r] += upd

    return pl.pallas_call(
        body,
        grid=(nblk,),
        in_specs=[
            pl.BlockSpec((BR, f), lambda i: (i, 0)),
            pl.BlockSpec((BR, NC * n_rel), lambda i: (i, 0)),
        ],
        out_specs=[
            pl.BlockSpec((n_rel, f, f), lambda i: (0, 0, 0)),
            pl.BlockSpec((n_rel, 8, f), lambda i: (0, 0, 0)),
        ],
        out_shape=[
            jax.ShapeDtypeStruct((n_rel, f, f), jnp.float32),
            jax.ShapeDtypeStruct((n_rel, 8, f), jnp.float32),
        ],
    )(x, cd)


def _tc_prepare(x, dd, s3, aux, wq, wk, wv, wp, *, n_rel, f):
    n = x.shape[0]
    nblk = n // BR

    def body(xb, db, s_ref, aux_ref, wq_ref, wk_ref, wv_ref, wp_ref,
             v3_ref, g_ref, dense_ref):
        xv = xb[...]
        dv = db[...]
        wp_ = wp_ref[...]
        ones_row = jnp.ones((1, f), jnp.float32)
        dense = jnp.zeros((BR, f), jnp.float32)
        for r in range(n_rel):
            wqr = wq_ref[r]
            wkr = wk_ref[r]
            wvr = wv_ref[r]
            s_mat = s_ref[r]
            sv = lax.dot_general(s_mat, wvr, (((1,), (0,)), ((), ())),
                                 preferred_element_type=jnp.float32)
            kvs = lax.dot_general(wkr, sv, (((0,), (0,)), ((), ())),
                                  preferred_element_type=jnp.float32)
            a2 = wqr @ kvs @ wp_
            s_row = aux_ref[r, 0:1, :]
            ks_row = lax.dot_general(s_row, wkr, (((1,), (0,)), ((), ())),
                                     preferred_element_type=jnp.float32)
            b_row = lax.dot_general(ks_row, wqr, (((1,), (1,)), ((), ())),
                                    preferred_element_type=jnp.float32)
            b_mat = lax.dot_general(b_row, ones_row, (((0,), (0,)), ((), ())),
                                    preferred_element_type=jnp.float32)
            cnt_row = aux_ref[r, 1:2, :]
            den = (xv @ b_mat) + cnt_row
            den = jnp.where(den == 0.0, 1.0, den)
            g_val = cnt_row / den
            g_ref[r] = g_val[:, 0:L]
            v3_ref[r] = xv @ (wvr @ wp_)
            dr = dv[:, r:r + 1] + dv[:, n_rel + r:n_rel + r + 1]
            dr128 = lax.dot_general(dr, ones_row, (((1,), (0,)), ((), ())),
                                    preferred_element_type=jnp.float32)
            dense = dense + (xv @ a2) * (dr128 / den)
        dense_ref[...] = dense

    return pl.pallas_call(
        body,
        grid=(nblk,),
        in_specs=[
            pl.BlockSpec((BR, f), lambda i: (i, 0)),
            pl.BlockSpec((BR, NC * n_rel), lambda i: (i, 0)),
            pl.BlockSpec((n_rel, f, f), lambda i: (0, 0, 0)),
            pl.BlockSpec((n_rel, 8, f), lambda i: (0, 0, 0)),
            pl.BlockSpec((n_rel, f, f), lambda i: (0, 0, 0)),
            pl.BlockSpec((n_rel, f, f), lambda i: (0, 0, 0)),
            pl.BlockSpec((n_rel, f, f), lambda i: (0, 0, 0)),
            pl.BlockSpec((f, f), lambda i: (0, 0)),
        ],
        out_specs=[
            pl.BlockSpec((n_rel, BR, f), lambda i: (0, i, 0)),
            pl.BlockSpec((n_rel, BR, L), lambda i: (0, i, 0)),
            pl.BlockSpec((BR, f), lambda i: (i, 0)),
        ],
        out_shape=[
            jax.ShapeDtypeStruct((n_rel, n, f), jnp.float32),
            jax.ShapeDtypeStruct((n_rel, n, L), jnp.float32),
            jax.ShapeDtypeStruct((n, f), jnp.float32),
        ],
    )(x, dd, s3, aux, wq, wk, wv, wp)


def _tc_combine(parts, dense, *, f):
    n = dense.shape[0]
    nblk = n // BR

    def body(p_ref, d_ref, o_ref):
        o_ref[...] = p_ref[0] + p_ref[1] + d_ref[...]

    return pl.pallas_call(
        body,
        grid=(nblk,),
        in_specs=[
            pl.BlockSpec((NC, BR, f), lambda i: (0, i, 0)),
            pl.BlockSpec((BR, f), lambda i: (i, 0)),
        ],
        out_specs=pl.BlockSpec((BR, f), lambda i: (i, 0)),
        out_shape=jax.ShapeDtypeStruct((n, f), jnp.float32),
    )(parts, dense)


def kernel(x, edge_index, edge_type, Wq, Wk, Wv, Wp):
    n, f = x.shape
    r_ = Wq.shape[0]
    e = edge_type.shape[0]
    assert e % NW == 0
    epw = e // NW
    assert epw % BATCH == 0 and f % L == 0 and n % BR == 0

    rn = r_ * n
    cnt_pad = ((2 * rn + NS * L - 1) // (NS * L)) * (NS * L)
    n_pad = ((n + NS * BATCH - 1) // (NS * BATCH)) * (NS * BATCH)
    assert r_ <= 4

    rowv = edge_index[0]
    colv = edge_index[1]
    typv = edge_type.astype(jnp.int32)

    cnts = _sc_counts(rowv, colv, typv,
                      n_nodes=n, n_rel=r_, epw=epw,
                      cnt_pad=cnt_pad).reshape(NC, cnt_pad)
    cd = cnts[:, :rn].reshape(NC * r_, n).T
    dd = cnts[:, rn:2 * rn].reshape(NC * r_, n).T

    s3, aux = _tc_reduce(x, cd, n_rel=r_, f=f)
    v3, g, dense = _tc_prepare(x, dd, s3, aux, Wq, Wk, Wv, Wp, n_rel=r_, f=f)

    parts = _sc_edge_pass(rowv, colv, typv,
                          v3.reshape(rn, f), g.reshape(rn, L),
                          n_nodes=n, epw=epw, n_pad=n_pad, f=f)

    return _tc_combine(parts, dense, f=f)
